# Initial kernel scaffold; baseline (speedup 1.0000x reference)
import jax
import jax.numpy as jnp
from jax.experimental import pallas as pl


def kernel(user_indices, item_indices, user_emb, item_emb, adj_row, adj_col, adj_val):
    raise NotImplementedError("write your pallas kernel here")


# speedup vs baseline: 3.0968x; 3.0968x over previous
import functools

import jax
import jax.numpy as jnp
from jax import lax
from jax.experimental import pallas as pl
from jax.experimental.pallas import tpu as pltpu
from jax.experimental.pallas import tpu_sc as plsc

NUM_USERS = 5000
N_NODES = 10000
D = 128
DB = D // 16

NC = 2
NS = 16
NW = NC * NS

E_TOTAL = 320000
CH = 80
EDGES_PER_TILE = E_TOTAL // NW
N_CHUNKS = EDGES_PER_TILE // CH

N_RCH = N_NODES // CH
N_RCH_PER_TILE = (N_RCH + NS - 1) // NS

B = 16384
B_PER_TILE = B // NW
BCH = 128

_mesh = plsc.VectorSubcoreMesh(
    core_axis_name="c", subcore_axis_name="s", num_cores=NC, num_subcores=NS
)

_f32 = jnp.float32
_i32 = jnp.int32


def _propagate_body(full_h, row_h, col_h, val_h, p0_h, p1_h,
                    acc, colv, rowv, valv, rowsv, sem):
    c = lax.axis_index("c")
    s = lax.axis_index("s")

    def init_chunk(k, carry):
        idx = s + k * NS

        @pl.when(idx < N_RCH)
        def _():
            base = idx * CH
            pltpu.sync_copy(full_h.at[pl.ds(base, CH)], rowsv)

            def scale(r, carry2):
                for j in range(DB):
                    rowsv[r, pl.ds(16 * j, 16)] = rowsv[r, pl.ds(16 * j, 16)] * 0.25
                return carry2

            lax.fori_loop(0, CH, scale, 0)
            pltpu.sync_copy(rowsv, acc.at[pl.ds(base, CH)])

        return carry

    lax.fori_loop(0, N_RCH_PER_TILE, init_chunk, 0)
    plsc.subcore_barrier()

    ebase = (c * NS + s) * EDGES_PER_TILE

    def edge_chunk(i, carry):
        base = ebase + i * CH
        pltpu.sync_copy(col_h.at[pl.ds(base, CH)], colv)
        pltpu.sync_copy(row_h.at[pl.ds(base, CH)], rowv)
        pltpu.sync_copy(val_h.at[pl.ds(base, CH)], valv)
        pltpu.async_copy(full_h.at[colv], rowsv, sem).wait()

        def scale_row(r, carry2):
            vb = plsc.load_gather(valv, [jnp.full((16,), r, _i32)]) * 0.5
            for j in range(DB):
                rowsv[r, pl.ds(16 * j, 16)] = rowsv[r, pl.ds(16 * j, 16)] * vb
            return carry2

        lax.fori_loop(0, CH, scale_row, 0)
        pltpu.sync_copy(rowsv, acc.at[rowv], add=True)
        return carry

    lax.fori_loop(0, N_CHUNKS, edge_chunk, 0)
    plsc.subcore_barrier()

    def wo_chunk(k, carry):
        idx = s + k * NS

        @pl.when(idx < N_RCH)
        def _():
            base = idx * CH

            @pl.when(c == 0)
            def _w0():
                pltpu.sync_copy(acc.at[pl.ds(base, CH)], p0_h.at[pl.ds(base, CH)])

            @pl.when(c == 1)
            def _w1():
                pltpu.sync_copy(acc.at[pl.ds(base, CH)], p1_h.at[pl.ds(base, CH)])

        return carry

    lax.fori_loop(0, N_RCH_PER_TILE, wo_chunk, 0)


_propagate = pl.kernel(
    _propagate_body,
    out_type=(
        jax.ShapeDtypeStruct((N_NODES, D), _f32),
        jax.ShapeDtypeStruct((N_NODES, D), _f32),
    ),
    mesh=_mesh,
    compiler_params=pltpu.CompilerParams(needs_layout_passes=False),
    scratch_types=[
        pltpu.VMEM_SHARED((N_NODES, D), _f32),
        pltpu.VMEM((CH,), _i32),
        pltpu.VMEM((CH,), _i32),
        pltpu.VMEM((CH,), _f32),
        pltpu.VMEM((CH, D), _f32),
        pltpu.SemaphoreType.DMA,
    ],
)


def _readout_body(uidx_h, iidx_h, p0_h, p1_h, out_h,
                  uix, iix, bu0, bu1, bi0, bi1, outv, sem):
    c = lax.axis_index("c")
    s = lax.axis_index("s")
    obase = (c * NS + s) * B_PER_TILE

    def chunk(cc, carry):
        base = obase + cc * BCH
        pltpu.sync_copy(uidx_h.at[pl.ds(base, BCH)], uix)
        pltpu.sync_copy(iidx_h.at[pl.ds(base, BCH)], iix)

        def shift(k, carry2):
            iix[pl.ds(k * 16, 16)] = iix[pl.ds(k * 16, 16)] + NUM_USERS
            return carry2

        lax.fori_loop(0, BCH // 16, shift, 0)

        cp0 = pltpu.async_copy(p0_h.at[uix], bu0, sem)
        cp1 = pltpu.async_copy(p1_h.at[uix], bu1, sem)
        cp2 = pltpu.async_copy(p0_h.at[iix], bi0, sem)
        cp3 = pltpu.async_copy(p1_h.at[iix], bi1, sem)
        cp0.wait()
        cp1.wait()
        cp2.wait()
        cp3.wait()

        def group(g, carry2):
            bvec = lax.iota(_i32, 16) + g * 16

            def dstep(d, accv):
                dvec = jnp.full((16,), d, _i32)
                uf = (plsc.load_gather(bu0, [bvec, dvec])
                      + plsc.load_gather(bu1, [bvec, dvec]))
                vf = (plsc.load_gather(bi0, [bvec, dvec])
                      + plsc.load_gather(bi1, [bvec, dvec]))
                return accv + uf * vf

            accv = lax.fori_loop(0, D, dstep, jnp.zeros((16,), _f32))
            outv[pl.ds(cc * BCH + g * 16, 16)] = accv
            return carry2

        lax.fori_loop(0, BCH // 16, group, 0)
        return carry

    lax.fori_loop(0, B_PER_TILE // BCH, chunk, 0)
    pltpu.sync_copy(outv, out_h.at[pl.ds(obase, B_PER_TILE)])


_readout = pl.kernel(
    _readout_body,
    out_type=jax.ShapeDtypeStruct((B,), _f32),
    mesh=_mesh,
    compiler_params=pltpu.CompilerParams(needs_layout_passes=False),
    scratch_types=[
        pltpu.VMEM((BCH,), _i32),
        pltpu.VMEM((BCH,), _i32),
        pltpu.VMEM((BCH, D), _f32),
        pltpu.VMEM((BCH, D), _f32),
        pltpu.VMEM((BCH, D), _f32),
        pltpu.VMEM((BCH, D), _f32),
        pltpu.VMEM((B_PER_TILE,), _f32),
        pltpu.SemaphoreType.DMA,
    ],
)


@jax.jit
def kernel(user_indices, item_indices, user_emb, item_emb, adj_row, adj_col, adj_val):
    full = jnp.concatenate([user_emb, item_emb], axis=0)
    p0, p1 = _propagate(full, adj_row, adj_col, adj_val)
    return _readout(user_indices, item_indices, p0, p1)


# speedup vs baseline: 6.0624x; 1.9577x over previous
import jax
import jax.numpy as jnp
from jax import lax
from jax.experimental import pallas as pl
from jax.experimental.pallas import tpu as pltpu
from jax.experimental.pallas import tpu_sc as plsc

NUM_USERS = 5000
N_NODES = 10000
D = 128
DB = D // 16

NC = 2
NS = 16
NW = NC * NS

E_TOTAL = 320000
CH = 40
EDGES_PER_TILE = E_TOTAL // NW
N_CHUNKS = EDGES_PER_TILE // CH

N_RCH = N_NODES // CH
N_RCH_PER_TILE = (N_RCH + NS - 1) // NS

B = 16384
B_PER_TILE = B // NW
BCH = 64
NBCH = B_PER_TILE // BCH

_mesh = plsc.VectorSubcoreMesh(
    core_axis_name="c", subcore_axis_name="s", num_cores=NC, num_subcores=NS
)

_f32 = jnp.float32
_i32 = jnp.int32


def _splat(x):
    return jnp.full((16,), x, _i32)


def _propagate_body(full_h, row_h, col_h, val_h, p0_h, p1_h,
                    acc, colv, valv, rowbuf, inb, outb, gsem, ssem):
    c = lax.axis_index("c")
    s = lax.axis_index("s")
    wid = c * NS + s
    ebase = wid * EDGES_PER_TILE

    def init_chunk(k, carry):
        idx = s + k * NS

        @pl.when(idx < N_RCH)
        def _():
            base = idx * CH
            pltpu.sync_copy(full_h.at[pl.ds(base, CH)], inb.at[0])

            @plsc.parallel_loop(0, CH, unroll=4)
            def _scale(r):
                for j in range(DB):
                    inb[0, r, pl.ds(16 * j, 16)] = inb[0, r, pl.ds(16 * j, 16)] * 0.25

            pltpu.sync_copy(inb.at[0], acc.at[pl.ds(base, CH)])

        return carry

    lax.fori_loop(0, N_RCH_PER_TILE, init_chunk, 0)
    plsc.subcore_barrier()

    pltpu.sync_copy(col_h.at[pl.ds(ebase, EDGES_PER_TILE)], colv)
    pltpu.sync_copy(val_h.at[pl.ds(ebase, EDGES_PER_TILE)], valv)

    def start_gather(k, b4):
        pltpu.async_copy(full_h.at[colv.at[pl.ds(k * CH, CH)]],
                         inb.at[b4 % 2], gsem.at[b4 % 2])
        pltpu.async_copy(row_h.at[pl.ds(ebase + k * CH, CH)],
                         rowbuf.at[b4], gsem.at[b4 % 2])

    def wait_gather(k, b4):
        pltpu.make_async_copy(full_h.at[colv.at[pl.ds(k * CH, CH)]],
                              inb.at[b4 % 2], gsem.at[b4 % 2]).wait()
        pltpu.make_async_copy(row_h.at[pl.ds(ebase + k * CH, CH)],
                              rowbuf.at[b4], gsem.at[b4 % 2]).wait()

    def start_scatter(k, b4):
        pltpu.async_copy(outb.at[b4 % 2], acc.at[rowbuf.at[b4]],
                         ssem.at[b4 % 2], add=True)

    def wait_scatter(bb):
        pltpu.make_async_copy(outb.at[bb], acc.at[pl.ds(0, CH)],
                              ssem.at[bb]).wait()

    def process(k, b4):
        bb = b4 % 2
        wait_gather(k, b4)

        @pl.when(k >= 2)
        def _():
            wait_scatter(bb)

        @plsc.parallel_loop(0, CH, unroll=4)
        def _scale(r):
            vb = plsc.load_gather(valv, [_splat(k * CH + r)]) * 0.5
            for j in range(DB):
                outb[bb, r, pl.ds(16 * j, 16)] = inb[bb, r, pl.ds(16 * j, 16)] * vb

        start_scatter(k, b4)

        @pl.when(k + 2 < N_CHUNKS)
        def _():
            start_gather(k + 2, (b4 + 2) % 4)

    start_gather(0, 0)
    start_gather(1, 1)

    @pl.loop(0, N_CHUNKS - 2, step=4)
    def _quad(i):
        for j in range(4):
            process(i + j, j)

    process(N_CHUNKS - 2, 0)
    process(N_CHUNKS - 1, 1)
    wait_scatter(0)
    wait_scatter(1)
    plsc.subcore_barrier()

    def wo_chunk(k, carry):
        idx = s + k * NS

        @pl.when(idx < N_RCH)
        def _():
            base = idx * CH

            @pl.when(c == 0)
            def _w0():
                pltpu.sync_copy(acc.at[pl.ds(base, CH)], p0_h.at[pl.ds(base, CH)])

            @pl.when(c == 1)
            def _w1():
                pltpu.sync_copy(acc.at[pl.ds(base, CH)], p1_h.at[pl.ds(base, CH)])

        return carry

    lax.fori_loop(0, N_RCH_PER_TILE, wo_chunk, 0)


_propagate = pl.kernel(
    _propagate_body,
    out_type=(
        jax.ShapeDtypeStruct((N_NODES, D), _f32),
        jax.ShapeDtypeStruct((N_NODES, D), _f32),
    ),
    mesh=_mesh,
    compiler_params=pltpu.CompilerParams(needs_layout_passes=False),
    scratch_types=[
        pltpu.VMEM_SHARED((N_NODES, D), _f32),
        pltpu.VMEM((EDGES_PER_TILE,), _i32),
        pltpu.VMEM((EDGES_PER_TILE,), _f32),
        pltpu.VMEM((4, CH), _i32),
        pltpu.VMEM((2, CH, D), _f32),
        pltpu.VMEM((2, CH, D), _f32),
        pltpu.SemaphoreType.DMA((2,)),
        pltpu.SemaphoreType.DMA((2,)),
    ],
)


def _readout_body(uidx_h, iidx_h, p0_h, p1_h, out_h,
                  uix, iix, bu0, bu1, bi0, bi1, outv, gsem):
    c = lax.axis_index("c")
    s = lax.axis_index("s")
    obase = (c * NS + s) * B_PER_TILE

    pltpu.sync_copy(uidx_h.at[pl.ds(obase, B_PER_TILE)], uix)
    pltpu.sync_copy(iidx_h.at[pl.ds(obase, B_PER_TILE)], iix)

    @plsc.parallel_loop(0, B_PER_TILE // 16, unroll=4)
    def _shift(k):
        iix[pl.ds(k * 16, 16)] = iix[pl.ds(k * 16, 16)] + NUM_USERS

    def start_gathers(cc, b):
        ui = uix.at[pl.ds(cc * BCH, BCH)]
        ii = iix.at[pl.ds(cc * BCH, BCH)]
        pltpu.async_copy(p0_h.at[ui], bu0.at[b], gsem.at[b])
        pltpu.async_copy(p1_h.at[ui], bu1.at[b], gsem.at[b])
        pltpu.async_copy(p0_h.at[ii], bi0.at[b], gsem.at[b])
        pltpu.async_copy(p1_h.at[ii], bi1.at[b], gsem.at[b])

    def wait_gathers(cc, b):
        ui = uix.at[pl.ds(cc * BCH, BCH)]
        ii = iix.at[pl.ds(cc * BCH, BCH)]
        pltpu.make_async_copy(p0_h.at[ui], bu0.at[b], gsem.at[b]).wait()
        pltpu.make_async_copy(p1_h.at[ui], bu1.at[b], gsem.at[b]).wait()
        pltpu.make_async_copy(p0_h.at[ii], bi0.at[b], gsem.at[b]).wait()
        pltpu.make_async_copy(p1_h.at[ii], bi1.at[b], gsem.at[b]).wait()

    def process(cc, b):
        wait_gathers(cc, b)

        def group(g, carry):
            bvec = lax.iota(_i32, 16) + g * 16

            @plsc.parallel_loop(0, D, unroll=4, carry=jnp.zeros((16,), _f32))
            def accv(d, a):
                dvec = _splat(d)
                uf = (plsc.load_gather(bu0.at[b], [bvec, dvec])
                      + plsc.load_gather(bu1.at[b], [bvec, dvec]))
                vf = (plsc.load_gather(bi0.at[b], [bvec, dvec])
                      + plsc.load_gather(bi1.at[b], [bvec, dvec]))
                return a + uf * vf

            outv[pl.ds(cc * BCH + g * 16, 16)] = accv
            return carry

        lax.fori_loop(0, BCH // 16, group, 0)

        @pl.when(cc + 2 < NBCH)
        def _():
            start_gathers(cc + 2, b)

    start_gathers(0, 0)
    start_gathers(1, 1)

    @pl.loop(0, NBCH, step=2)
    def _pair(i):
        process(i, 0)
        process(i + 1, 1)

    pltpu.sync_copy(outv, out_h.at[pl.ds(obase, B_PER_TILE)])


_readout = pl.kernel(
    _readout_body,
    out_type=jax.ShapeDtypeStruct((B,), _f32),
    mesh=_mesh,
    compiler_params=pltpu.CompilerParams(needs_layout_passes=False),
    scratch_types=[
        pltpu.VMEM((B_PER_TILE,), _i32),
        pltpu.VMEM((B_PER_TILE,), _i32),
        pltpu.VMEM((2, BCH, D), _f32),
        pltpu.VMEM((2, BCH, D), _f32),
        pltpu.VMEM((2, BCH, D), _f32),
        pltpu.VMEM((2, BCH, D), _f32),
        pltpu.VMEM((B_PER_TILE,), _f32),
        pltpu.SemaphoreType.DMA((2,)),
    ],
)


@jax.jit
def kernel(user_indices, item_indices, user_emb, item_emb, adj_row, adj_col, adj_val):
    full = jnp.concatenate([user_emb, item_emb], axis=0)
    p0, p1 = _propagate(full, adj_row, adj_col, adj_val)
    return _readout(user_indices, item_indices, p0, p1)


# speedup vs baseline: 8.5706x; 1.4137x over previous
import jax
import jax.numpy as jnp
from jax import lax
from jax.experimental import pallas as pl
from jax.experimental.pallas import tpu as pltpu
from jax.experimental.pallas import tpu_sc as plsc

NUM_USERS = 5000
N_NODES = 10000
D = 128
DB = D // 16

NC = 2
NS = 16
NW = NC * NS

E_TOTAL = 320000
CH = 40
EDGES_PER_TILE = E_TOTAL // NW
N_CHUNKS = EDGES_PER_TILE // CH

N_RCH = N_NODES // CH
N_RCH_PER_TILE = (N_RCH + NS - 1) // NS

B = 16384
B_PER_TILE = B // NW
BCH = 64
NBCH = B_PER_TILE // BCH

_mesh = plsc.VectorSubcoreMesh(
    core_axis_name="c", subcore_axis_name="s", num_cores=NC, num_subcores=NS
)

_f32 = jnp.float32
_i32 = jnp.int32


def _splat(x):
    return jnp.full((16,), x, _i32)


def _propagate_body(full_h, row_h, col_h, val_h, p0_h, p1_h,
                    acc, colv, valv, rowbuf, inb, outb, gsem, ssem):
    c = lax.axis_index("c")
    s = lax.axis_index("s")
    wid = c * NS + s
    ebase = wid * EDGES_PER_TILE

    @pl.when(c == 1)
    def _zero_staging():
        @plsc.parallel_loop(0, CH, unroll=4)
        def _z(r):
            for j in range(DB):
                inb[0, r, pl.ds(16 * j, 16)] = jnp.zeros((16,), _f32)

    def init_chunk(k, carry):
        idx = s + k * NS

        @pl.when(idx < N_RCH)
        def _():
            base = idx * CH

            @pl.when(c == 0)
            def _c0():
                pltpu.sync_copy(full_h.at[pl.ds(base, CH)], acc.at[pl.ds(base, CH)])

            @pl.when(c == 1)
            def _c1():
                pltpu.sync_copy(inb.at[0], acc.at[pl.ds(base, CH)])

        return carry

    lax.fori_loop(0, N_RCH_PER_TILE, init_chunk, 0)
    plsc.subcore_barrier()

    pltpu.sync_copy(col_h.at[pl.ds(ebase, EDGES_PER_TILE)], colv)
    pltpu.sync_copy(val_h.at[pl.ds(ebase, EDGES_PER_TILE)], valv)

    def start_gather(k, b4):
        pltpu.async_copy(full_h.at[colv.at[pl.ds(k * CH, CH)]],
                         inb.at[b4 % 2], gsem.at[b4 % 2])
        pltpu.async_copy(row_h.at[pl.ds(ebase + k * CH, CH)],
                         rowbuf.at[b4], gsem.at[b4 % 2])

    def wait_gather(k, b4):
        pltpu.make_async_copy(full_h.at[colv.at[pl.ds(k * CH, CH)]],
                              inb.at[b4 % 2], gsem.at[b4 % 2]).wait()
        pltpu.make_async_copy(row_h.at[pl.ds(ebase + k * CH, CH)],
                              rowbuf.at[b4], gsem.at[b4 % 2]).wait()

    def start_scatter(k, b4):
        pltpu.async_copy(outb.at[b4 % 2], acc.at[rowbuf.at[b4]],
                         ssem.at[b4 % 2], add=True)

    def wait_scatter(bb):
        pltpu.make_async_copy(outb.at[bb], acc.at[pl.ds(0, CH)],
                              ssem.at[bb]).wait()

    def process(k, b4):
        bb = b4 % 2
        wait_gather(k, b4)

        @pl.when(k >= 2)
        def _():
            wait_scatter(bb)

        @plsc.parallel_loop(0, CH, unroll=4)
        def _scale(r):
            vb = plsc.load_gather(valv, [_splat(k * CH + r)]) * 0.5
            for j in range(DB):
                outb[bb, r, pl.ds(16 * j, 16)] = inb[bb, r, pl.ds(16 * j, 16)] * vb

        start_scatter(k, b4)

        @pl.when(k + 2 < N_CHUNKS)
        def _():
            start_gather(k + 2, (b4 + 2) % 4)

    start_gather(0, 0)
    start_gather(1, 1)

    @pl.loop(0, N_CHUNKS - 2, step=4)
    def _quad(i):
        for j in range(4):
            process(i + j, j)

    process(N_CHUNKS - 2, 0)
    process(N_CHUNKS - 1, 1)
    wait_scatter(0)
    wait_scatter(1)
    plsc.subcore_barrier()

    def wo_chunk(k, carry):
        idx = s + k * NS

        @pl.when(idx < N_RCH)
        def _():
            base = idx * CH

            @pl.when(c == 0)
            def _w0():
                pltpu.sync_copy(acc.at[pl.ds(base, CH)], p0_h.at[pl.ds(base, CH)])

            @pl.when(c == 1)
            def _w1():
                pltpu.sync_copy(acc.at[pl.ds(base, CH)], p1_h.at[pl.ds(base, CH)])

        return carry

    lax.fori_loop(0, N_RCH_PER_TILE, wo_chunk, 0)


_propagate = pl.kernel(
    _propagate_body,
    out_type=(
        jax.ShapeDtypeStruct((N_NODES, D), _f32),
        jax.ShapeDtypeStruct((N_NODES, D), _f32),
    ),
    mesh=_mesh,
    compiler_params=pltpu.CompilerParams(needs_layout_passes=False),
    scratch_types=[
        pltpu.VMEM_SHARED((N_NODES, D), _f32),
        pltpu.VMEM((EDGES_PER_TILE,), _i32),
        pltpu.VMEM((EDGES_PER_TILE,), _f32),
        pltpu.VMEM((4, CH), _i32),
        pltpu.VMEM((2, CH, D), _f32),
        pltpu.VMEM((2, CH, D), _f32),
        pltpu.SemaphoreType.DMA((2,)),
        pltpu.SemaphoreType.DMA((2,)),
    ],
)


def _gather_rows_body(uidx_h, iidx_h, p0_h, p1_h, urows_h, irows_h,
                      uix, iix, bu0, bu1, bi0, bi1, ub, ib, gsem, wsem):
    c = lax.axis_index("c")
    s = lax.axis_index("s")
    obase = (c * NS + s) * B_PER_TILE

    pltpu.sync_copy(uidx_h.at[pl.ds(obase, B_PER_TILE)], uix)
    pltpu.sync_copy(iidx_h.at[pl.ds(obase, B_PER_TILE)], iix)

    @plsc.parallel_loop(0, B_PER_TILE // 16, unroll=4)
    def _shift(k):
        iix[pl.ds(k * 16, 16)] = iix[pl.ds(k * 16, 16)] + NUM_USERS

    def start_gathers(cc, b):
        ui = uix.at[pl.ds(cc * BCH, BCH)]
        ii = iix.at[pl.ds(cc * BCH, BCH)]
        pltpu.async_copy(p0_h.at[ui], bu0.at[b], gsem.at[b])
        pltpu.async_copy(p1_h.at[ui], bu1.at[b], gsem.at[b])
        pltpu.async_copy(p0_h.at[ii], bi0.at[b], gsem.at[b])
        pltpu.async_copy(p1_h.at[ii], bi1.at[b], gsem.at[b])

    def wait_gathers(cc, b):
        ui = uix.at[pl.ds(cc * BCH, BCH)]
        ii = iix.at[pl.ds(cc * BCH, BCH)]
        pltpu.make_async_copy(p0_h.at[ui], bu0.at[b], gsem.at[b]).wait()
        pltpu.make_async_copy(p1_h.at[ui], bu1.at[b], gsem.at[b]).wait()
        pltpu.make_async_copy(p0_h.at[ii], bi0.at[b], gsem.at[b]).wait()
        pltpu.make_async_copy(p1_h.at[ii], bi1.at[b], gsem.at[b]).wait()

    def start_writes(cc, b):
        base = obase + cc * BCH
        pltpu.async_copy(ub.at[b], urows_h.at[pl.ds(base, BCH)], wsem.at[b])
        pltpu.async_copy(ib.at[b], irows_h.at[pl.ds(base, BCH)], wsem.at[b])

    def wait_writes(cc, b):
        base = obase + cc * BCH
        pltpu.make_async_copy(ub.at[b], urows_h.at[pl.ds(base, BCH)],
                              wsem.at[b]).wait()
        pltpu.make_async_copy(ib.at[b], irows_h.at[pl.ds(base, BCH)],
                              wsem.at[b]).wait()

    def process(cc, b):
        wait_gathers(cc, b)

        @pl.when(cc >= 2)
        def _():
            wait_writes(cc - 2, b)

        @plsc.parallel_loop(0, BCH, unroll=2)
        def _add(r):
            for j in range(DB):
                ub[b, r, pl.ds(16 * j, 16)] = (bu0[b, r, pl.ds(16 * j, 16)]
                                               + bu1[b, r, pl.ds(16 * j, 16)])
                ib[b, r, pl.ds(16 * j, 16)] = (bi0[b, r, pl.ds(16 * j, 16)]
                                               + bi1[b, r, pl.ds(16 * j, 16)])

        start_writes(cc, b)

        @pl.when(cc + 2 < NBCH)
        def _():
            start_gathers(cc + 2, b)

    start_gathers(0, 0)
    start_gathers(1, 1)

    @pl.loop(0, NBCH, step=2)
    def _pair(i):
        process(i, 0)
        process(i + 1, 1)

    wait_writes(NBCH - 2, 0)
    wait_writes(NBCH - 1, 1)


_gather_rows = pl.kernel(
    _gather_rows_body,
    out_type=(
        jax.ShapeDtypeStruct((B, D), _f32),
        jax.ShapeDtypeStruct((B, D), _f32),
    ),
    mesh=_mesh,
    compiler_params=pltpu.CompilerParams(needs_layout_passes=False),
    scratch_types=[
        pltpu.VMEM((B_PER_TILE,), _i32),
        pltpu.VMEM((B_PER_TILE,), _i32),
        pltpu.VMEM((2, BCH, D), _f32),
        pltpu.VMEM((2, BCH, D), _f32),
        pltpu.VMEM((2, BCH, D), _f32),
        pltpu.VMEM((2, BCH, D), _f32),
        pltpu.VMEM((2, BCH, D), _f32),
        pltpu.VMEM((2, BCH, D), _f32),
        pltpu.SemaphoreType.DMA((2,)),
        pltpu.SemaphoreType.DMA((2,)),
    ],
)

_DOT_BLK = 1024


def _dot_body(u_ref, i_ref, o_ref):
    o_ref[...] = 0.25 * jnp.sum(u_ref[...] * i_ref[...], axis=1, keepdims=True)


_dot = pl.pallas_call(
    _dot_body,
    grid=(B // _DOT_BLK,),
    in_specs=[
        pl.BlockSpec((_DOT_BLK, D), lambda i: (i, 0)),
        pl.BlockSpec((_DOT_BLK, D), lambda i: (i, 0)),
    ],
    out_specs=pl.BlockSpec((_DOT_BLK, 1), lambda i: (i, 0)),
    out_shape=jax.ShapeDtypeStruct((B, 1), _f32),
)


@jax.jit
def kernel(user_indices, item_indices, user_emb, item_emb, adj_row, adj_col, adj_val):
    full = jnp.concatenate([user_emb, item_emb], axis=0)
    p0, p1 = _propagate(full, adj_row, adj_col, adj_val)
    urows, irows = _gather_rows(user_indices, item_indices, p0, p1)
    return _dot(urows, irows).reshape(B)


# speedup vs baseline: 8.5812x; 1.0012x over previous
import jax
import jax.numpy as jnp
from jax import lax
from jax.experimental import pallas as pl
from jax.experimental.pallas import tpu as pltpu
from jax.experimental.pallas import tpu_sc as plsc

NUM_USERS = 5000
N_NODES = 10000
D = 128
DB = D // 16

NC = 2
NS = 16
NW = NC * NS

E_TOTAL = 320000
CH = 40
EDGES_PER_TILE = E_TOTAL // NW
N_CHUNKS = EDGES_PER_TILE // CH

N_RCH = N_NODES // CH
N_RCH_PER_TILE = (N_RCH + NS - 1) // NS

B = 16384
B_PER_TILE = B // NW
BCH = 64
NBCH = B_PER_TILE // BCH

_mesh = plsc.VectorSubcoreMesh(
    core_axis_name="c", subcore_axis_name="s", num_cores=NC, num_subcores=NS
)

_f32 = jnp.float32
_i32 = jnp.int32


def _splat(x):
    return jnp.full((16,), x, _i32)


def _propagate_body(full_h, row_h, col_h, val_h, p0_h, p1_h,
                    acc, colv, valv, rowbuf, inb, outb, gsem, ssem):
    c = lax.axis_index("c")
    s = lax.axis_index("s")
    wid = c * NS + s
    ebase = wid * EDGES_PER_TILE

    @pl.when(c == 1)
    def _zero_staging():
        @plsc.parallel_loop(0, CH, unroll=4)
        def _z(r):
            for j in range(DB):
                inb[0, r, pl.ds(16 * j, 16)] = jnp.zeros((16,), _f32)

    def init_chunk(k, carry):
        idx = s + k * NS

        @pl.when(idx < N_RCH)
        def _():
            base = idx * CH

            @pl.when(c == 0)
            def _c0():
                pltpu.sync_copy(full_h.at[pl.ds(base, CH)], acc.at[pl.ds(base, CH)])

            @pl.when(c == 1)
            def _c1():
                pltpu.sync_copy(inb.at[0], acc.at[pl.ds(base, CH)])

        return carry

    lax.fori_loop(0, N_RCH_PER_TILE, init_chunk, 0)
    plsc.subcore_barrier()

    pltpu.sync_copy(col_h.at[pl.ds(ebase, EDGES_PER_TILE)], colv)
    pltpu.sync_copy(val_h.at[pl.ds(ebase, EDGES_PER_TILE)], valv)

    def start_gather(k, b4):
        pltpu.async_copy(full_h.at[colv.at[pl.ds(k * CH, CH)]],
                         inb.at[b4 % 2], gsem.at[b4 % 2])
        pltpu.async_copy(row_h.at[pl.ds(ebase + k * CH, CH)],
                         rowbuf.at[b4], gsem.at[b4 % 2])

    def wait_gather(k, b4):
        pltpu.make_async_copy(full_h.at[colv.at[pl.ds(k * CH, CH)]],
                              inb.at[b4 % 2], gsem.at[b4 % 2]).wait()
        pltpu.make_async_copy(row_h.at[pl.ds(ebase + k * CH, CH)],
                              rowbuf.at[b4], gsem.at[b4 % 2]).wait()

    def start_scatter(k, b4):
        pltpu.async_copy(outb.at[b4 % 2], acc.at[rowbuf.at[b4]],
                         ssem.at[b4 % 2], add=True)

    def wait_scatter(bb):
        pltpu.make_async_copy(outb.at[bb], acc.at[pl.ds(0, CH)],
                              ssem.at[bb]).wait()

    def process(k, b4):
        bb = b4 % 2
        wait_gather(k, b4)

        @pl.when(k >= 2)
        def _():
            wait_scatter(bb)

        @plsc.parallel_loop(0, CH, unroll=4)
        def _scale(r):
            vb = plsc.load_gather(valv, [_splat(k * CH + r)])
            for j in range(DB):
                outb[bb, r, pl.ds(16 * j, 16)] = inb[bb, r, pl.ds(16 * j, 16)] * vb

        start_scatter(k, b4)

        @pl.when(k + 2 < N_CHUNKS)
        def _():
            start_gather(k + 2, (b4 + 2) % 4)

    start_gather(0, 0)
    start_gather(1, 1)

    @pl.loop(0, N_CHUNKS - 2, step=4)
    def _quad(i):
        for j in range(4):
            process(i + j, j)

    process(N_CHUNKS - 2, 0)
    process(N_CHUNKS - 1, 1)
    wait_scatter(0)
    wait_scatter(1)
    plsc.subcore_barrier()

    def wo_chunk(k, carry):
        idx = s + k * NS

        @pl.when(idx < N_RCH)
        def _():
            base = idx * CH

            @pl.when(c == 0)
            def _w0():
                pltpu.sync_copy(acc.at[pl.ds(base, CH)], p0_h.at[pl.ds(base, CH)])

            @pl.when(c == 1)
            def _w1():
                pltpu.sync_copy(acc.at[pl.ds(base, CH)], p1_h.at[pl.ds(base, CH)])

        return carry

    lax.fori_loop(0, N_RCH_PER_TILE, wo_chunk, 0)


_propagate = pl.kernel(
    _propagate_body,
    out_type=(
        jax.ShapeDtypeStruct((N_NODES, D), _f32),
        jax.ShapeDtypeStruct((N_NODES, D), _f32),
    ),
    mesh=_mesh,
    compiler_params=pltpu.CompilerParams(needs_layout_passes=False),
    scratch_types=[
        pltpu.VMEM_SHARED((N_NODES, D), _f32),
        pltpu.VMEM((EDGES_PER_TILE,), _i32),
        pltpu.VMEM((EDGES_PER_TILE,), _f32),
        pltpu.VMEM((4, CH), _i32),
        pltpu.VMEM((2, CH, D), _f32),
        pltpu.VMEM((2, CH, D), _f32),
        pltpu.SemaphoreType.DMA((2,)),
        pltpu.SemaphoreType.DMA((2,)),
    ],
)


def _gather_rows_body(uidx_h, iidx_h, p0_h, p1_h, urows_h, irows_h,
                      uix, iix, bu0, bu1, bi0, bi1, ub, ib, gsem, wsem):
    c = lax.axis_index("c")
    s = lax.axis_index("s")
    obase = (c * NS + s) * B_PER_TILE

    pltpu.sync_copy(uidx_h.at[pl.ds(obase, B_PER_TILE)], uix)
    pltpu.sync_copy(iidx_h.at[pl.ds(obase, B_PER_TILE)], iix)

    @plsc.parallel_loop(0, B_PER_TILE // 16, unroll=4)
    def _shift(k):
        iix[pl.ds(k * 16, 16)] = iix[pl.ds(k * 16, 16)] + NUM_USERS

    def start_gathers(cc, b):
        ui = uix.at[pl.ds(cc * BCH, BCH)]
        ii = iix.at[pl.ds(cc * BCH, BCH)]
        pltpu.async_copy(p0_h.at[ui], bu0.at[b], gsem.at[b])
        pltpu.async_copy(p1_h.at[ui], bu1.at[b], gsem.at[b])
        pltpu.async_copy(p0_h.at[ii], bi0.at[b], gsem.at[b])
        pltpu.async_copy(p1_h.at[ii], bi1.at[b], gsem.at[b])

    def wait_gathers(cc, b):
        ui = uix.at[pl.ds(cc * BCH, BCH)]
        ii = iix.at[pl.ds(cc * BCH, BCH)]
        pltpu.make_async_copy(p0_h.at[ui], bu0.at[b], gsem.at[b]).wait()
        pltpu.make_async_copy(p1_h.at[ui], bu1.at[b], gsem.at[b]).wait()
        pltpu.make_async_copy(p0_h.at[ii], bi0.at[b], gsem.at[b]).wait()
        pltpu.make_async_copy(p1_h.at[ii], bi1.at[b], gsem.at[b]).wait()

    def start_writes(cc, b):
        base = obase + cc * BCH
        pltpu.async_copy(ub.at[b], urows_h.at[pl.ds(base, BCH)], wsem.at[b])
        pltpu.async_copy(ib.at[b], irows_h.at[pl.ds(base, BCH)], wsem.at[b])

    def wait_writes(cc, b):
        base = obase + cc * BCH
        pltpu.make_async_copy(ub.at[b], urows_h.at[pl.ds(base, BCH)],
                              wsem.at[b]).wait()
        pltpu.make_async_copy(ib.at[b], irows_h.at[pl.ds(base, BCH)],
                              wsem.at[b]).wait()

    def process(cc, b):
        wait_gathers(cc, b)

        @pl.when(cc >= 2)
        def _():
            wait_writes(cc - 2, b)

        @plsc.parallel_loop(0, BCH, unroll=2)
        def _add(r):
            for j in range(DB):
                ub[b, r, pl.ds(16 * j, 16)] = (bu0[b, r, pl.ds(16 * j, 16)]
                                               + bu1[b, r, pl.ds(16 * j, 16)])
                ib[b, r, pl.ds(16 * j, 16)] = (bi0[b, r, pl.ds(16 * j, 16)]
                                               + bi1[b, r, pl.ds(16 * j, 16)])

        start_writes(cc, b)

        @pl.when(cc + 2 < NBCH)
        def _():
            start_gathers(cc + 2, b)

    start_gathers(0, 0)
    start_gathers(1, 1)

    @pl.loop(0, NBCH, step=2)
    def _pair(i):
        process(i, 0)
        process(i + 1, 1)

    wait_writes(NBCH - 2, 0)
    wait_writes(NBCH - 1, 1)


_gather_rows = pl.kernel(
    _gather_rows_body,
    out_type=(
        jax.ShapeDtypeStruct((B, D), _f32),
        jax.ShapeDtypeStruct((B, D), _f32),
    ),
    mesh=_mesh,
    compiler_params=pltpu.CompilerParams(needs_layout_passes=False),
    scratch_types=[
        pltpu.VMEM((B_PER_TILE,), _i32),
        pltpu.VMEM((B_PER_TILE,), _i32),
        pltpu.VMEM((2, BCH, D), _f32),
        pltpu.VMEM((2, BCH, D), _f32),
        pltpu.VMEM((2, BCH, D), _f32),
        pltpu.VMEM((2, BCH, D), _f32),
        pltpu.VMEM((2, BCH, D), _f32),
        pltpu.VMEM((2, BCH, D), _f32),
        pltpu.SemaphoreType.DMA((2,)),
        pltpu.SemaphoreType.DMA((2,)),
    ],
)

_DOT_BLK = 1024


def _dot_body(u_ref, i_ref, o_ref):
    o_ref[...] = 0.25 * jnp.sum(u_ref[...] * i_ref[...], axis=1, keepdims=True)


_dot = pl.pallas_call(
    _dot_body,
    grid=(B // _DOT_BLK,),
    in_specs=[
        pl.BlockSpec((_DOT_BLK, D), lambda i: (i, 0)),
        pl.BlockSpec((_DOT_BLK, D), lambda i: (i, 0)),
    ],
    out_specs=pl.BlockSpec((_DOT_BLK, 1), lambda i: (i, 0)),
    out_shape=jax.ShapeDtypeStruct((B, 1), _f32),
)


@jax.jit
def kernel(user_indices, item_indices, user_emb, item_emb, adj_row, adj_col, adj_val):
    full = jnp.concatenate([user_emb, item_emb], axis=0)
    p0, p1 = _propagate(full, adj_row, adj_col, adj_val)
    urows, irows = _gather_rows(user_indices, item_indices, p0, p1)
    return _dot(urows, irows).reshape(B)


# speedup vs baseline: 9.7681x; 1.1383x over previous
import jax
import jax.numpy as jnp
from jax import lax
from jax.experimental import pallas as pl
from jax.experimental.pallas import tpu as pltpu
from jax.experimental.pallas import tpu_sc as plsc

NUM_USERS = 5000
N_NODES = 10000
D = 128
DB = D // 16

NC = 2
NS = 16
NW = NC * NS

E_TOTAL = 320000
CH = 80
EDGES_PER_TILE = E_TOTAL // NW
N_CHUNKS = EDGES_PER_TILE // CH

N_RCH = N_NODES // CH
N_RCH_PER_TILE = (N_RCH + NS - 1) // NS

B = 16384
B_PER_TILE = B // NW
BCH = 64
NBCH = B_PER_TILE // BCH

_mesh = plsc.VectorSubcoreMesh(
    core_axis_name="c", subcore_axis_name="s", num_cores=NC, num_subcores=NS
)

_f32 = jnp.float32
_i32 = jnp.int32


def _splat(x):
    return jnp.full((16,), x, _i32)


def _propagate_body(full_h, row_h, col_h, val_h, p0_h, p1_h,
                    acc, colbuf, rowbuf, valbuf, inb, outb,
                    lsem, gsem, ssem):
    c = lax.axis_index("c")
    s = lax.axis_index("s")
    wid = c * NS + s
    ebase = wid * EDGES_PER_TILE

    @pl.when(c == 1)
    def _zero_staging():
        @plsc.parallel_loop(0, CH, unroll=4)
        def _z(r):
            for j in range(DB):
                inb[0, r, pl.ds(16 * j, 16)] = jnp.zeros((16,), _f32)

    def init_chunk(k, carry):
        idx = s + k * NS

        @pl.when(idx < N_RCH)
        def _():
            base = idx * CH

            @pl.when(c == 0)
            def _c0():
                pltpu.sync_copy(full_h.at[pl.ds(base, CH)],
                                acc.at[pl.ds(base, CH)])

            @pl.when(c == 1)
            def _c1():
                pltpu.sync_copy(inb.at[0], acc.at[pl.ds(base, CH)])

        return carry

    lax.fori_loop(0, N_RCH_PER_TILE, init_chunk, 0)
    plsc.subcore_barrier()

    def edge_slice(k):
        return pl.ds(ebase + k * CH, CH)

    def start_idx(k, b4, bb):
        pltpu.async_copy(col_h.at[edge_slice(k)], colbuf.at[b4], lsem.at[bb])
        pltpu.async_copy(row_h.at[edge_slice(k)], rowbuf.at[b4], lsem.at[bb])
        pltpu.async_copy(val_h.at[edge_slice(k)], valbuf.at[b4], lsem.at[bb])

    def wait_idx(k, b4, bb):
        pltpu.make_async_copy(col_h.at[edge_slice(k)], colbuf.at[b4],
                              lsem.at[bb]).wait()
        pltpu.make_async_copy(row_h.at[edge_slice(k)], rowbuf.at[b4],
                              lsem.at[bb]).wait()
        pltpu.make_async_copy(val_h.at[edge_slice(k)], valbuf.at[b4],
                              lsem.at[bb]).wait()

    def start_g(b4, bb):
        pltpu.async_copy(full_h.at[colbuf.at[b4]], inb.at[bb], gsem.at[bb])

    def wait_g(b4, bb):
        pltpu.make_async_copy(full_h.at[colbuf.at[b4]], inb.at[bb],
                              gsem.at[bb]).wait()

    def start_w(b4, bb):
        pltpu.async_copy(outb.at[bb], acc.at[rowbuf.at[b4]], ssem.at[bb],
                         add=True)

    def wait_w(bb):
        pltpu.make_async_copy(outb.at[bb], acc.at[pl.ds(0, CH)],
                              ssem.at[bb]).wait()

    def process(k, b4):
        bb = b4 % 2

        @pl.when(k + 1 < N_CHUNKS)
        def _():
            wait_idx(k + 1, (b4 + 1) % 4, 1 - bb)
            start_g((b4 + 1) % 4, 1 - bb)

        wait_g(b4, bb)

        @pl.when(k >= 2)
        def _():
            wait_w(bb)

        @pl.when(k + 2 < N_CHUNKS)
        def _():
            start_idx(k + 2, (b4 + 2) % 4, bb)

        @plsc.parallel_loop(0, CH, unroll=4)
        def _scale(r):
            vb = plsc.load_gather(valbuf, [_splat(b4), _splat(r)])
            for j in range(DB):
                outb[bb, r, pl.ds(16 * j, 16)] = inb[bb, r, pl.ds(16 * j, 16)] * vb

        start_w(b4, bb)

    start_idx(0, 0, 0)
    start_idx(1, 1, 1)
    wait_idx(0, 0, 0)
    start_g(0, 0)

    @pl.loop(0, N_CHUNKS - 1, step=4)
    def _quad(i):
        for j in range(4):
            process(i + j, j)

    process(N_CHUNKS - 1, 0)
    wait_w(0)
    wait_w(1)
    plsc.subcore_barrier()

    def wo_chunk(k, carry):
        idx = s + k * NS

        @pl.when(idx < N_RCH)
        def _():
            base = idx * CH

            @pl.when(c == 0)
            def _w0():
                pltpu.sync_copy(acc.at[pl.ds(base, CH)],
                                p0_h.at[pl.ds(base, CH)])

            @pl.when(c == 1)
            def _w1():
                pltpu.sync_copy(acc.at[pl.ds(base, CH)],
                                p1_h.at[pl.ds(base, CH)])

        return carry

    lax.fori_loop(0, N_RCH_PER_TILE, wo_chunk, 0)


_propagate = pl.kernel(
    _propagate_body,
    out_type=(
        jax.ShapeDtypeStruct((N_NODES, D), _f32),
        jax.ShapeDtypeStruct((N_NODES, D), _f32),
    ),
    mesh=_mesh,
    compiler_params=pltpu.CompilerParams(needs_layout_passes=False),
    scratch_types=[
        pltpu.VMEM_SHARED((N_NODES, D), _f32),
        pltpu.VMEM((4, CH), _i32),
        pltpu.VMEM((4, CH), _i32),
        pltpu.VMEM((4, CH), _f32),
        pltpu.VMEM((2, CH, D), _f32),
        pltpu.VMEM((2, CH, D), _f32),
        pltpu.SemaphoreType.DMA((2,)),
        pltpu.SemaphoreType.DMA((2,)),
        pltpu.SemaphoreType.DMA((2,)),
    ],
)


def _gather_rows_body(uidx_h, iidx_h, p0_h, p1_h, urows_h, irows_h,
                      uix, iix, bu0, bu1, bi0, bi1, ub, ib, gsem, wsem):
    c = lax.axis_index("c")
    s = lax.axis_index("s")
    obase = (c * NS + s) * B_PER_TILE

    pltpu.sync_copy(uidx_h.at[pl.ds(obase, B_PER_TILE)], uix)
    pltpu.sync_copy(iidx_h.at[pl.ds(obase, B_PER_TILE)], iix)

    @plsc.parallel_loop(0, B_PER_TILE // 16, unroll=4)
    def _shift(k):
        iix[pl.ds(k * 16, 16)] = iix[pl.ds(k * 16, 16)] + NUM_USERS

    def start_gathers(cc, b):
        ui = uix.at[pl.ds(cc * BCH, BCH)]
        ii = iix.at[pl.ds(cc * BCH, BCH)]
        pltpu.async_copy(p0_h.at[ui], bu0.at[b], gsem.at[b])
        pltpu.async_copy(p1_h.at[ui], bu1.at[b], gsem.at[b])
        pltpu.async_copy(p0_h.at[ii], bi0.at[b], gsem.at[b])
        pltpu.async_copy(p1_h.at[ii], bi1.at[b], gsem.at[b])

    def wait_gathers(cc, b):
        ui = uix.at[pl.ds(cc * BCH, BCH)]
        ii = iix.at[pl.ds(cc * BCH, BCH)]
        pltpu.make_async_copy(p0_h.at[ui], bu0.at[b], gsem.at[b]).wait()
        pltpu.make_async_copy(p1_h.at[ui], bu1.at[b], gsem.at[b]).wait()
        pltpu.make_async_copy(p0_h.at[ii], bi0.at[b], gsem.at[b]).wait()
        pltpu.make_async_copy(p1_h.at[ii], bi1.at[b], gsem.at[b]).wait()

    def start_writes(cc, b):
        base = obase + cc * BCH
        pltpu.async_copy(ub.at[b], urows_h.at[pl.ds(base, BCH)], wsem.at[b])
        pltpu.async_copy(ib.at[b], irows_h.at[pl.ds(base, BCH)], wsem.at[b])

    def wait_writes(cc, b):
        base = obase + cc * BCH
        pltpu.make_async_copy(ub.at[b], urows_h.at[pl.ds(base, BCH)],
                              wsem.at[b]).wait()
        pltpu.make_async_copy(ib.at[b], irows_h.at[pl.ds(base, BCH)],
                              wsem.at[b]).wait()

    def process(cc, b):
        wait_gathers(cc, b)

        @pl.when(cc >= 2)
        def _():
            wait_writes(cc - 2, b)

        @plsc.parallel_loop(0, BCH, unroll=2)
        def _add(r):
            for j in range(DB):
                ub[b, r, pl.ds(16 * j, 16)] = (bu0[b, r, pl.ds(16 * j, 16)]
                                               + bu1[b, r, pl.ds(16 * j, 16)])
                ib[b, r, pl.ds(16 * j, 16)] = (bi0[b, r, pl.ds(16 * j, 16)]
                                               + bi1[b, r, pl.ds(16 * j, 16)])

        start_writes(cc, b)

        @pl.when(cc + 2 < NBCH)
        def _():
            start_gathers(cc + 2, b)

    start_gathers(0, 0)
    start_gathers(1, 1)

    @pl.loop(0, NBCH, step=2)
    def _pair(i):
        process(i, 0)
        process(i + 1, 1)

    wait_writes(NBCH - 2, 0)
    wait_writes(NBCH - 1, 1)


_gather_rows = pl.kernel(
    _gather_rows_body,
    out_type=(
        jax.ShapeDtypeStruct((B, D), _f32),
        jax.ShapeDtypeStruct((B, D), _f32),
    ),
    mesh=_mesh,
    compiler_params=pltpu.CompilerParams(needs_layout_passes=False),
    scratch_types=[
        pltpu.VMEM((B_PER_TILE,), _i32),
        pltpu.VMEM((B_PER_TILE,), _i32),
        pltpu.VMEM((2, BCH, D), _f32),
        pltpu.VMEM((2, BCH, D), _f32),
        pltpu.VMEM((2, BCH, D), _f32),
        pltpu.VMEM((2, BCH, D), _f32),
        pltpu.VMEM((2, BCH, D), _f32),
        pltpu.VMEM((2, BCH, D), _f32),
        pltpu.SemaphoreType.DMA((2,)),
        pltpu.SemaphoreType.DMA((2,)),
    ],
)

_DOT_BLK = 1024


def _dot_body(u_ref, i_ref, o_ref):
    o_ref[...] = 0.25 * jnp.sum(u_ref[...] * i_ref[...], axis=1, keepdims=True)


_dot = pl.pallas_call(
    _dot_body,
    grid=(B // _DOT_BLK,),
    in_specs=[
        pl.BlockSpec((_DOT_BLK, D), lambda i: (i, 0)),
        pl.BlockSpec((_DOT_BLK, D), lambda i: (i, 0)),
    ],
    out_specs=pl.BlockSpec((_DOT_BLK, 1), lambda i: (i, 0)),
    out_shape=jax.ShapeDtypeStruct((B, 1), _f32),
)


@jax.jit
def kernel(user_indices, item_indices, user_emb, item_emb, adj_row, adj_col, adj_val):
    full = jnp.concatenate([user_emb, item_emb], axis=0)
    p0, p1 = _propagate(full, adj_row, adj_col, adj_val)
    urows, irows = _gather_rows(user_indices, item_indices, p0, p1)
    return _dot(urows, irows).reshape(B)


# speedup vs baseline: 10.9679x; 1.1228x over previous
import jax
import jax.numpy as jnp
from jax import lax
from jax.experimental import pallas as pl
from jax.experimental.pallas import tpu as pltpu
from jax.experimental.pallas import tpu_sc as plsc

NUM_USERS = 5000
N_NODES = 10000
D = 128
DB = D // 16

NC = 2
NS = 16
NW = NC * NS

E_TOTAL = 320000
CH = 80
EDGES_PER_TILE = E_TOTAL // NW
N_CHUNKS = EDGES_PER_TILE // CH

N_RCH = N_NODES // CH
N_RCH_PER_TILE = (N_RCH + NS - 1) // NS

B = 16384
B_PER_TILE = B // NW
BCH = 64
NBCH = B_PER_TILE // BCH

_mesh = plsc.VectorSubcoreMesh(
    core_axis_name="c", subcore_axis_name="s", num_cores=NC, num_subcores=NS
)

_f32 = jnp.float32
_i32 = jnp.int32


def _splat(x):
    return jnp.full((16,), x, _i32)


def _propagate_body(full_h, row_h, col_h, val_h, p0_h, p1_h,
                    acc, colbuf, rowbuf, valbuf, inb, outb,
                    lsem, gsem, ssem):
    c = lax.axis_index("c")
    s = lax.axis_index("s")
    wid = c * NS + s
    ebase = wid * EDGES_PER_TILE

    @pl.when(c == 1)
    def _zero_staging():
        @plsc.parallel_loop(0, CH, unroll=4)
        def _z(r):
            for j in range(DB):
                inb[0, r, pl.ds(16 * j, 16)] = jnp.zeros((16,), _f32)

    def init_chunk(k, carry):
        idx = s + k * NS

        @pl.when(idx < N_RCH)
        def _():
            base = idx * CH

            @pl.when(c == 0)
            def _c0():
                pltpu.sync_copy(full_h.at[pl.ds(base, CH)],
                                acc.at[pl.ds(base, CH)])

            @pl.when(c == 1)
            def _c1():
                pltpu.sync_copy(inb.at[0], acc.at[pl.ds(base, CH)])

        return carry

    lax.fori_loop(0, N_RCH_PER_TILE, init_chunk, 0)
    plsc.subcore_barrier()

    def edge_slice(k):
        return pl.ds(ebase + k * CH, CH)

    def start_idx(k, b4, bb):
        pltpu.async_copy(col_h.at[edge_slice(k)], colbuf.at[b4], lsem.at[bb])
        pltpu.async_copy(row_h.at[edge_slice(k)], rowbuf.at[b4], lsem.at[bb])
        pltpu.async_copy(val_h.at[edge_slice(k)], valbuf.at[b4], lsem.at[bb])

    def wait_idx(k, b4, bb):
        pltpu.make_async_copy(col_h.at[edge_slice(k)], colbuf.at[b4],
                              lsem.at[bb]).wait()
        pltpu.make_async_copy(row_h.at[edge_slice(k)], rowbuf.at[b4],
                              lsem.at[bb]).wait()
        pltpu.make_async_copy(val_h.at[edge_slice(k)], valbuf.at[b4],
                              lsem.at[bb]).wait()

    def start_g(b4, bb):
        pltpu.async_copy(full_h.at[colbuf.at[b4]], inb.at[bb], gsem.at[bb])

    def wait_g(b4, bb):
        pltpu.make_async_copy(full_h.at[colbuf.at[b4]], inb.at[bb],
                              gsem.at[bb]).wait()

    def start_w(b4, bb):
        pltpu.async_copy(outb.at[bb], acc.at[rowbuf.at[b4]], ssem.at[bb],
                         add=True)

    def wait_w(bb):
        pltpu.make_async_copy(outb.at[bb], acc.at[pl.ds(0, CH)],
                              ssem.at[bb]).wait()

    def process(k, b4):
        bb = b4 % 2

        @pl.when(k + 1 < N_CHUNKS)
        def _():
            wait_idx(k + 1, (b4 + 1) % 4, 1 - bb)
            start_g((b4 + 1) % 4, 1 - bb)

        wait_g(b4, bb)

        @pl.when(k >= 2)
        def _():
            wait_w(bb)

        @pl.when(k + 2 < N_CHUNKS)
        def _():
            start_idx(k + 2, (b4 + 2) % 4, bb)

        @plsc.parallel_loop(0, CH, unroll=4)
        def _scale(r):
            vb = plsc.load_gather(valbuf, [_splat(b4), _splat(r)])
            for j in range(DB):
                outb[bb, r, pl.ds(16 * j, 16)] = inb[bb, r, pl.ds(16 * j, 16)] * vb

        start_w(b4, bb)

    start_idx(0, 0, 0)
    start_idx(1, 1, 1)
    wait_idx(0, 0, 0)
    start_g(0, 0)

    @pl.loop(0, N_CHUNKS - 1, step=4)
    def _quad(i):
        for j in range(4):
            process(i + j, j)

    process(N_CHUNKS - 1, 0)
    wait_w(0)
    wait_w(1)
    plsc.subcore_barrier()

    def wo_chunk(k, carry):
        idx = s + k * NS

        @pl.when(idx < N_RCH)
        def _():
            base = idx * CH

            @pl.when(c == 0)
            def _w0():
                pltpu.sync_copy(acc.at[pl.ds(base, CH)],
                                p0_h.at[pl.ds(base, CH)])

            @pl.when(c == 1)
            def _w1():
                pltpu.sync_copy(acc.at[pl.ds(base, CH)],
                                p1_h.at[pl.ds(base, CH)])

        return carry

    lax.fori_loop(0, N_RCH_PER_TILE, wo_chunk, 0)


_propagate = pl.kernel(
    _propagate_body,
    out_type=(
        jax.ShapeDtypeStruct((N_NODES, D), _f32),
        jax.ShapeDtypeStruct((N_NODES, D), _f32),
    ),
    mesh=_mesh,
    compiler_params=pltpu.CompilerParams(needs_layout_passes=False),
    scratch_types=[
        pltpu.VMEM_SHARED((N_NODES, D), _f32),
        pltpu.VMEM((4, CH), _i32),
        pltpu.VMEM((4, CH), _i32),
        pltpu.VMEM((4, CH), _f32),
        pltpu.VMEM((2, CH, D), _f32),
        pltpu.VMEM((2, CH, D), _f32),
        pltpu.SemaphoreType.DMA((2,)),
        pltpu.SemaphoreType.DMA((2,)),
        pltpu.SemaphoreType.DMA((2,)),
    ],
)


def _readout_body(uidx_h, iidx_h, p0_h, p1_h, out_h,
                  uix, iix, bu0, bu1, bi0, bi1, pbuf, outv, gsem):
    c = lax.axis_index("c")
    s = lax.axis_index("s")
    obase = (c * NS + s) * B_PER_TILE

    pltpu.sync_copy(uidx_h.at[pl.ds(obase, B_PER_TILE)], uix)
    pltpu.sync_copy(iidx_h.at[pl.ds(obase, B_PER_TILE)], iix)

    @plsc.parallel_loop(0, B_PER_TILE // 16, unroll=4)
    def _shift(k):
        iix[pl.ds(k * 16, 16)] = iix[pl.ds(k * 16, 16)] + NUM_USERS

    def start_gathers(cc, b):
        ui = uix.at[pl.ds(cc * BCH, BCH)]
        ii = iix.at[pl.ds(cc * BCH, BCH)]
        pltpu.async_copy(p0_h.at[ui], bu0.at[b], gsem.at[b])
        pltpu.async_copy(p1_h.at[ui], bu1.at[b], gsem.at[b])
        pltpu.async_copy(p0_h.at[ii], bi0.at[b], gsem.at[b])
        pltpu.async_copy(p1_h.at[ii], bi1.at[b], gsem.at[b])

    def wait_gathers(cc, b):
        ui = uix.at[pl.ds(cc * BCH, BCH)]
        ii = iix.at[pl.ds(cc * BCH, BCH)]
        pltpu.make_async_copy(p0_h.at[ui], bu0.at[b], gsem.at[b]).wait()
        pltpu.make_async_copy(p1_h.at[ui], bu1.at[b], gsem.at[b]).wait()
        pltpu.make_async_copy(p0_h.at[ii], bi0.at[b], gsem.at[b]).wait()
        pltpu.make_async_copy(p1_h.at[ii], bi1.at[b], gsem.at[b]).wait()

    def process(cc, b):
        wait_gathers(cc, b)

        @pl.when(cc + 2 < NBCH)
        def _():
            start_gathers(cc + 2, b)

        def group(g, carry):
            @plsc.parallel_loop(0, 16, unroll=2)
            def _el(e):
                r = g * 16 + e
                p = jnp.zeros((16,), _f32)
                for j in range(DB):
                    u = (bu0[b, r, pl.ds(16 * j, 16)]
                         + bu1[b, r, pl.ds(16 * j, 16)])
                    v = (bi0[b, r, pl.ds(16 * j, 16)]
                         + bi1[b, r, pl.ds(16 * j, 16)])
                    p = p + u * v
                pbuf[e, pl.ds(0, 16)] = p

            bvec = lax.iota(_i32, 16)

            @plsc.parallel_loop(0, 16, carry=jnp.zeros((16,), _f32))
            def acc16(j, a):
                return a + plsc.load_gather(pbuf, [bvec, _splat(j)])

            outv[pl.ds(cc * BCH + g * 16, 16)] = acc16 * 0.25
            return carry

        lax.fori_loop(0, BCH // 16, group, 0)

    start_gathers(0, 0)
    start_gathers(1, 1)

    @pl.loop(0, NBCH, step=2)
    def _pair(i):
        process(i, 0)
        process(i + 1, 1)

    pltpu.sync_copy(outv, out_h.at[pl.ds(obase, B_PER_TILE)])


_readout = pl.kernel(
    _readout_body,
    out_type=jax.ShapeDtypeStruct((B,), _f32),
    mesh=_mesh,
    compiler_params=pltpu.CompilerParams(needs_layout_passes=False),
    scratch_types=[
        pltpu.VMEM((B_PER_TILE,), _i32),
        pltpu.VMEM((B_PER_TILE,), _i32),
        pltpu.VMEM((2, BCH, D), _f32),
        pltpu.VMEM((2, BCH, D), _f32),
        pltpu.VMEM((2, BCH, D), _f32),
        pltpu.VMEM((2, BCH, D), _f32),
        pltpu.VMEM((16, 16), _f32),
        pltpu.VMEM((B_PER_TILE,), _f32),
        pltpu.SemaphoreType.DMA((2,)),
    ],
)


@jax.jit
def kernel(user_indices, item_indices, user_emb, item_emb, adj_row, adj_col, adj_val):
    full = jnp.concatenate([user_emb, item_emb], axis=0)
    p0, p1 = _propagate(full, adj_row, adj_col, adj_val)
    return _readout(user_indices, item_indices, p0, p1)
